# baseline (speedup 1.0000x reference)
import functools

import jax
import jax.numpy as jnp
import numpy as np
from jax import lax
from jax.experimental import pallas as pl
from jax.experimental.pallas import tpu as pltpu
from jax.experimental.pallas import tpu_sc as plsc

D_MODEL = 256
NUM_HEAD = 8
NUM_LEVEL = 4
KK = 4
HEAD_DIM = 32
LEVEL_SHAPES = [(64, 64), (32, 32), (16, 16), (8, 8)]
B = 4
L1 = 1024
L2 = 5440
NQ = B * L1
NITEM = NQ * NUM_HEAD
NROW = NITEM * 64


def _by_level(l_lane, vals, dtype):
    out = jnp.full(l_lane.shape, vals[3], dtype)
    for lv in (2, 1, 0):
        out = jnp.where(l_lane == lv, jnp.asarray(vals[lv], dtype), out)
    return out


def _vproj_body(x_ref, w_ref, b_ref, mf_ref, o_ref):
    v = jnp.dot(x_ref[...], w_ref[...].T, preferred_element_type=jnp.float32,
                 precision=lax.Precision.HIGHEST)
    o_ref[...] = ((v + b_ref[...]) * mf_ref[...]).astype(jnp.bfloat16)


def _qside_body(q_ref, rw_ref, aw_ref, ab_ref, bw_ref, bb_ref, rx_ref, ry_ref,
                sp_ref, lv_ref, i0_ref, i1_ref, i2_ref, i3_ref,
                w0_ref, w1_ref, w2_ref, w3_ref):
    f32 = jnp.float32
    i32 = jnp.int32
    li = lax.broadcasted_iota(i32, (1, 128), 1)
    l_lane = (li % 16) // 4
    k_lane = li % 4
    hlane = li // 16
    ri = lax.broadcasted_iota(i32, (128, 128), 0)
    ci = lax.broadcasted_iota(i32, (128, 128), 1)
    gs = (ri // 16 == ci // 16).astype(f32)
    gl = ((ri // 16 == ci // 16) & (ri % 4 == ci % 4)).astype(f32)
    psel = [((ri // 4 == ci // 4) & (ri % 4 == c)).astype(f32)
            for c in range(4)]
    r4 = lax.broadcasted_iota(i32, (4, 128), 0)
    c4 = lax.broadcasted_iota(i32, (4, 128), 1)
    t1 = (r4 == c4 % 4).astype(f32)
    t2 = (r4 == 2 + (c4 % 4) % 2).astype(f32)

    q = q_ref[0]
    aw = jnp.dot(q, aw_ref[...].T, preferred_element_type=f32,
                 precision=lax.Precision.HIGHEST) + ab_ref[...]
    rowmax = jnp.max(aw, axis=1, keepdims=True)
    e = jnp.exp(aw - rowmax)
    sp_ref[0] = e / jnp.dot(e, gs, preferred_element_type=f32,
                 precision=lax.Precision.HIGHEST)
    lv_ref[0] = e / jnp.dot(e, gl, preferred_element_type=f32,
                 precision=lax.Precision.HIGHEST)

    ob = jnp.dot(q, bw_ref[...].T, preferred_element_type=f32,
                 precision=lax.Precision.HIGHEST) + bb_ref[...]
    rw = rw_ref[0]
    rw_c = jnp.dot(rw, t1, preferred_element_type=f32,
                 precision=lax.Precision.HIGHEST)
    rw_s = jnp.dot(rw, t2, preferred_element_type=f32,
                 precision=lax.Precision.HIGHEST)
    boxes = rw_c + ob * 0.125 * rw_s
    cx = jnp.dot(boxes, psel[0], preferred_element_type=f32,
                 precision=lax.Precision.HIGHEST)
    cy = jnp.dot(boxes, psel[1], preferred_element_type=f32,
                 precision=lax.Precision.HIGHEST)
    bw_sz = jnp.dot(boxes, psel[2], preferred_element_type=f32,
                 precision=lax.Precision.HIGHEST)
    bh_sz = jnp.dot(boxes, psel[3], preferred_element_type=f32,
                 precision=lax.Precision.HIGHEST)

    xoff = jnp.where(k_lane % 2 == 0, -0.25, 0.25).astype(f32)
    yoff = jnp.where(k_lane // 2 == 0, -0.25, 0.25).astype(f32)
    wlf = _by_level(l_lane, [64.0, 32.0, 16.0, 8.0], f32)
    hlf = wlf
    wli = _by_level(l_lane, [64, 32, 16, 8], i32)
    startl = _by_level(l_lane, [0, 4096, 5120, 5376], i32)

    gx = (cx + xoff * jnp.maximum(bw_sz, 0.0)) * rx_ref[0]
    gy = (cy + yoff * jnp.maximum(bh_sz, 0.0)) * ry_ref[0]
    x = gx * wlf - 0.5
    y = gy * hlf - 0.5
    x0 = jnp.floor(x)
    y0 = jnp.floor(y)
    fx = x - x0
    fy = y - y0

    boff = pl.program_id(0) * L2

    idx_refs = (i0_ref, i1_ref, i2_ref, i3_ref)
    w_refs = (w0_ref, w1_ref, w2_ref, w3_ref)
    corners = ((0.0, 0.0), (1.0, 0.0), (0.0, 1.0), (1.0, 1.0))
    for c, (dx, dy) in enumerate(corners):
        xc = x0 + dx
        yc = y0 + dy
        valid = ((xc >= 0.0) & (xc <= wlf - 1.0)
                 & (yc >= 0.0) & (yc <= hlf - 1.0))
        ii = jnp.clip(xc, 0.0, wlf - 1.0).astype(jnp.int32)
        jj = jnp.clip(yc, 0.0, hlf - 1.0).astype(jnp.int32)
        pos = startl + jj * wli + ii
        idx_refs[c][0] = (boff + pos) * NUM_HEAD + hlane
        wx = fx if dx == 1.0 else 1.0 - fx
        wy = fy if dy == 1.0 else 1.0 - fy
        w_refs[c][0] = wx * wy * valid.astype(jnp.float32)


def _oproj_body(x_ref, w_ref, b_ref, o_ref):
    o_ref[...] = jnp.dot(x_ref[...], w_ref[...].T,
                         preferred_element_type=jnp.float32,
                 precision=lax.Precision.HIGHEST) + b_ref[...]


_NC = 2
_NS = 16
_NW = _NC * _NS
_QPT = NQ // _NW
_QPC = 4
_NCHUNK = _QPT // _QPC


def _splat(vec, lane):
    dnums = lax.GatherDimensionNumbers(
        offset_dims=(), collapsed_slice_dims=(0,), start_index_map=(0,))
    idx = jnp.full((16, 1), lane, jnp.int32)
    return lax.gather(vec, idx, dnums, (1,),
                      mode=lax.GatherScatterMode.PROMISE_IN_BOUNDS)


_QPS = 16
_NSUPER = _QPT // _QPS
_CPS = _QPS // _QPC


def _sc_body(table, i0, i1, i2, i3, w0, w1, w2, w3, sp_hbm, lv_hbm,
             out_hbm, mask_hbm,
             idx_sv, wb_sv, sp_sv, lv_sv, rows_v, out_sv, mask_sv,
             sem0, sem1, msem0, msem1):
    wid = lax.axis_index("s") * _NC + lax.axis_index("c")
    q_tile = wid * _QPT
    idx_arrs = (i0, i1, i2, i3)
    wb_arrs = (w0, w1, w2, w3)
    sems = (sem0, sem1)
    msems = (msem0, msem1)

    def meta_copies(g, ms):
        qs0 = q_tile + g * _QPS
        cps = []
        for c in range(4):
            cps.append((idx_arrs[c].at[pl.ds(qs0, _QPS)],
                        idx_sv.at[pl.ds(ms * 64 + c * _QPS, _QPS)]))
            cps.append((wb_arrs[c].at[pl.ds(qs0 * 128, _QPS * 128)],
                        wb_sv.at[pl.ds(ms * 8192 + c * 2048, 2048)]))
        cps.append((sp_hbm.at[pl.ds(qs0 * 128, _QPS * 128)],
                    sp_sv.at[pl.ds(ms * 2048, 2048)]))
        cps.append((lv_hbm.at[pl.ds(qs0 * 128, _QPS * 128)],
                    lv_sv.at[pl.ds(ms * 2048, 2048)]))
        return cps

    def fire_meta(g, ms):
        for src, dst in meta_copies(g, ms):
            pltpu.async_copy(src, dst, msems[ms])

    def wait_meta(g, ms):
        for src, dst in meta_copies(g, ms):
            pltpu.make_async_copy(src, dst, msems[ms]).wait()

    def fire(j, slot, ms):
        for c in range(4):
            for q in range(_QPC):
                ql = j * _QPC + q
                pltpu.async_copy(
                    table.at[idx_sv.at[ms * 64 + c * _QPS + ql]],
                    rows_v.at[pl.ds(slot * 2048 + (q * 4 + c) * 128, 128)],
                    sems[slot])

    def wait_slot(slot):
        pltpu.make_async_copy(table.at[pl.ds(0, 2048)],
                              rows_v.at[pl.ds(slot * 2048, 2048)],
                              sems[slot]).wait()

    def compute(j, slot, ms):
        def t_body(t, c2):
            q = t // NUM_HEAD
            h = t % NUM_HEAD
            ql = j * _QPC + q
            lo = pl.ds(0, 16)
            hi = pl.ds(16, 16)
            spvec = sp_sv[pl.ds(ms * 2048 + ql * 128 + h * 16, 16)]
            lvvec = lv_sv[pl.ds(ms * 2048 + ql * 128 + h * 16, 16)]
            wvecs = [wb_sv[pl.ds(ms * 8192 + c * 2048 + ql * 128 + h * 16,
                                 16)]
                     for c in range(4)]
            out0 = jnp.zeros((16,), jnp.float32)
            out1 = jnp.zeros((16,), jnp.float32)
            for k in range(4):
                m0 = jnp.zeros((16,), jnp.float32)
                m1 = jnp.zeros((16,), jnp.float32)
                for l in range(4):
                    lk = l * 4 + k
                    s0 = jnp.zeros((16,), jnp.float32)
                    s1 = jnp.zeros((16,), jnp.float32)
                    for c in range(4):
                        rr = slot * 2048 + (q * 4 + c) * 128 + h * 16 + lk
                        w = _splat(wvecs[c], lk)
                        ve, vo = plsc.unpack(
                            rows_v[rr, :],
                            format=plsc.PackFormat.INTERLEAVED)
                        s0 = s0 + w * ve
                        s1 = s1 + w * vo
                    sp = _splat(spvec, lk)
                    lv = _splat(lvvec, lk)
                    out0 = out0 + sp * s0
                    out1 = out1 + sp * s1
                    m0 = m0 + lv * s0
                    m1 = m1 + lv * s1
                mrow = ql * 32 + k * 8 + h
                mask_sv[mrow, lo] = m0
                mask_sv[mrow, hi] = m1
            orow = ql * 8 + h
            out_sv[orow, lo] = out0
            out_sv[orow, hi] = out1
            return c2
        lax.fori_loop(0, _QPC * NUM_HEAD, t_body, 0)

    fire_meta(0, 0)

    def spair_body(gp, carry):
        for ms in range(2):
            g = 2 * gp + ms
            qs0 = q_tile + g * _QPS
            wait_meta(g, ms)

            @pl.when(g < _NSUPER - 1)
            def _():
                fire_meta(g + 1, 1 - ms)

            fire(0, 0, ms)

            def pair_body(p, c2):
                j0 = 2 * p
                fire(j0 + 1, 1, ms)
                wait_slot(0)
                compute(j0, 0, ms)

                @pl.when(p < _CPS // 2 - 1)
                def _():
                    fire(j0 + 2, 0, ms)

                wait_slot(1)
                compute(j0 + 1, 1, ms)
                return c2
            lax.fori_loop(0, _CPS // 2, pair_body, 0)

            pltpu.sync_copy(out_sv, out_hbm.at[pl.ds(qs0 * 8, _QPS * 8)])
            pltpu.sync_copy(mask_sv,
                            mask_hbm.at[pl.ds(qs0 * 32, _QPS * 32)])
        return carry

    lax.fori_loop(0, _NSUPER // 2, spair_body, 0)


@functools.lru_cache(maxsize=1)
def _build_sc():
    return pl.kernel(
        _sc_body,
        out_type=(jax.ShapeDtypeStruct((NITEM, HEAD_DIM), jnp.float32),
                  jax.ShapeDtypeStruct((NITEM * 4, HEAD_DIM), jnp.float32)),
        mesh=plsc.VectorSubcoreMesh(core_axis_name="c", subcore_axis_name="s",
                                    num_cores=_NC, num_subcores=_NS),
        compiler_params=pltpu.CompilerParams(use_tc_tiling_on_sc=False,
                                             needs_layout_passes=False),
        scratch_types=[
            pltpu.VMEM((2 * 4 * _QPS, 128), jnp.int32),
            pltpu.VMEM((2 * 4 * _QPS * 128,), jnp.float32),
            pltpu.VMEM((2 * _QPS * 128,), jnp.float32),
            pltpu.VMEM((2 * _QPS * 128,), jnp.float32),
            pltpu.VMEM((2 * 2048, HEAD_DIM), jnp.bfloat16),
            pltpu.VMEM((_QPS * 8, HEAD_DIM), jnp.float32),
            pltpu.VMEM((_QPS * 32, HEAD_DIM), jnp.float32),
            pltpu.SemaphoreType.DMA,
            pltpu.SemaphoreType.DMA,
            pltpu.SemaphoreType.DMA,
            pltpu.SemaphoreType.DMA,
        ],
    )


def _sc_call(table, idxs, wbs, sp_flat, lv_flat):
    return _build_sc()(table, *idxs, *wbs, sp_flat, lv_flat)


def kernel(query, value, v_shape, v_mask, v_start_index, v_valid_ratios,
           ref_windows, value_proj_w, value_proj_b, out_proj_w, out_proj_b,
           box_w, box_b, attn_w, attn_b):
    f32 = jnp.float32

    maskf = (1.0 - v_mask.astype(f32)).reshape(B * L2, 1)
    vproj = pl.pallas_call(
        _vproj_body,
        grid=(8,),
        in_specs=[
            pl.BlockSpec((B * L2 // 8, D_MODEL), lambda i: (i, 0)),
            pl.BlockSpec((D_MODEL, D_MODEL), lambda i: (0, 0)),
            pl.BlockSpec((1, D_MODEL), lambda i: (0, 0)),
            pl.BlockSpec((B * L2 // 8, 1), lambda i: (i, 0)),
        ],
        out_specs=pl.BlockSpec((B * L2 // 8, D_MODEL), lambda i: (i, 0)),
        out_shape=jax.ShapeDtypeStruct((B * L2, D_MODEL), jnp.bfloat16),
    )(value.reshape(B * L2, D_MODEL), value_proj_w,
      value_proj_b.reshape(1, D_MODEL), maskf)
    table = vproj.reshape(B * L2 * NUM_HEAD, HEAD_DIM)

    hd_perm = jnp.asarray(
        np.concatenate([h * 32 + np.concatenate([np.arange(0, 32, 2),
                                                 np.arange(1, 32, 2)])
                        for h in range(NUM_HEAD)]).astype(np.int32))
    out_proj_w_p = out_proj_w[:, hd_perm]

    r = v_valid_ratios[:, 0, 0, :, 0, :]
    rx = jnp.broadcast_to(r[:, None, :, None, 0], (B, 8, 4, 4)).reshape(B, 1, 128)
    ry = jnp.broadcast_to(r[:, None, :, None, 1], (B, 8, 4, 4)).reshape(B, 1, 128)
    qout = pl.pallas_call(
        _qside_body,
        grid=(B,),
        in_specs=[
            pl.BlockSpec((1, L1, D_MODEL), lambda b: (b, 0, 0)),
            pl.BlockSpec((1, L1, 4), lambda b: (b, 0, 0)),
            pl.BlockSpec((128, D_MODEL), lambda b: (0, 0)),
            pl.BlockSpec((1, 128), lambda b: (0, 0)),
            pl.BlockSpec((128, D_MODEL), lambda b: (0, 0)),
            pl.BlockSpec((1, 128), lambda b: (0, 0)),
            pl.BlockSpec((1, 1, 128), lambda b: (b, 0, 0)),
            pl.BlockSpec((1, 1, 128), lambda b: (b, 0, 0)),
        ],
        out_specs=[pl.BlockSpec((1, L1, 128), lambda b: (b, 0, 0))] * 10,
        out_shape=[jax.ShapeDtypeStruct((B, L1, 128), f32),
                   jax.ShapeDtypeStruct((B, L1, 128), f32)]
                  + [jax.ShapeDtypeStruct((B, L1, 128), jnp.int32)] * 4
                  + [jax.ShapeDtypeStruct((B, L1, 128), f32)] * 4,
    )(query, ref_windows, attn_w, attn_b.reshape(1, 128),
      box_w, box_b.reshape(1, 128), rx, ry)
    spatial_l, level_l, i0, i1, i2, i3, w0, w1, w2, w3 = qout

    out_s, mask_s = _sc_call(
        table,
        [i.reshape(NQ, 128) for i in (i0, i1, i2, i3)],
        [w.reshape(-1) for w in (w0, w1, w2, w3)],
        spatial_l.reshape(-1), level_l.reshape(-1))

    def oproj(x, rows_per_block, nblocks):
        return pl.pallas_call(
            _oproj_body,
            grid=(nblocks,),
            in_specs=[
                pl.BlockSpec((rows_per_block, D_MODEL), lambda i: (i, 0)),
                pl.BlockSpec((D_MODEL, D_MODEL), lambda i: (0, 0)),
                pl.BlockSpec((1, D_MODEL), lambda i: (0, 0)),
            ],
            out_specs=pl.BlockSpec((rows_per_block, D_MODEL), lambda i: (i, 0)),
            out_shape=jax.ShapeDtypeStruct((x.shape[0], D_MODEL), f32),
        )(x, out_proj_w_p, out_proj_b.reshape(1, D_MODEL))

    out = oproj(out_s.reshape(NQ, D_MODEL), 2048, 2).reshape(B, L1, D_MODEL)
    mask_out = oproj(mask_s.reshape(NQ * 4, D_MODEL), 2048, 8)
    mask_out = mask_out.reshape(B, L1, KK, D_MODEL)

    spatial = spatial_l.reshape(B, L1, NUM_HEAD, NUM_LEVEL, KK)
    level = level_l.reshape(B, L1, NUM_HEAD, NUM_LEVEL, KK)
    return (out, mask_out, spatial, level)


# speedup vs baseline: 1.0591x; 1.0591x over previous
import functools

import jax
import jax.numpy as jnp
import numpy as np
from jax import lax
from jax.experimental import pallas as pl
from jax.experimental.pallas import tpu as pltpu
from jax.experimental.pallas import tpu_sc as plsc

D_MODEL = 256
NUM_HEAD = 8
NUM_LEVEL = 4
KK = 4
HEAD_DIM = 32
LEVEL_SHAPES = [(64, 64), (32, 32), (16, 16), (8, 8)]
B = 4
L1 = 1024
L2 = 5440
NQ = B * L1
NITEM = NQ * NUM_HEAD
NROW = NITEM * 64


def _by_level(l_lane, vals, dtype):
    out = jnp.full(l_lane.shape, vals[3], dtype)
    for lv in (2, 1, 0):
        out = jnp.where(l_lane == lv, jnp.asarray(vals[lv], dtype), out)
    return out


def _vproj_body(x_ref, w_ref, b_ref, mf_ref, o_ref):
    v = jnp.dot(x_ref[...], w_ref[...].T, preferred_element_type=jnp.float32,
                 precision=lax.Precision.HIGHEST)
    o_ref[...] = ((v + b_ref[...]) * mf_ref[...]).astype(jnp.bfloat16)


def _qside_body(q_ref, rw_ref, aw_ref, ab_ref, bw_ref, bb_ref, rx_ref, ry_ref,
                sp_ref, lv_ref, i0_ref, i1_ref, i2_ref, i3_ref,
                w0_ref, w1_ref, w2_ref, w3_ref):
    f32 = jnp.float32
    i32 = jnp.int32
    li = lax.broadcasted_iota(i32, (1, 128), 1)
    l_lane = (li % 16) // 4
    k_lane = li % 4
    hlane = li // 16
    ri = lax.broadcasted_iota(i32, (128, 128), 0)
    ci = lax.broadcasted_iota(i32, (128, 128), 1)
    gs = (ri // 16 == ci // 16).astype(f32)
    gl = ((ri // 16 == ci // 16) & (ri % 4 == ci % 4)).astype(f32)
    psel = [((ri // 4 == ci // 4) & (ri % 4 == c)).astype(f32)
            for c in range(4)]
    r4 = lax.broadcasted_iota(i32, (4, 128), 0)
    c4 = lax.broadcasted_iota(i32, (4, 128), 1)
    t1 = (r4 == c4 % 4).astype(f32)
    t2 = (r4 == 2 + (c4 % 4) % 2).astype(f32)

    q = q_ref[0]
    aw = jnp.dot(q, aw_ref[...].T, preferred_element_type=f32,
                 precision=lax.Precision.HIGHEST) + ab_ref[...]
    rowmax = jnp.max(aw, axis=1, keepdims=True)
    e = jnp.exp(aw - rowmax)
    sp_ref[0] = e / jnp.dot(e, gs, preferred_element_type=f32,
                 precision=lax.Precision.HIGHEST)
    lv_ref[0] = e / jnp.dot(e, gl, preferred_element_type=f32,
                 precision=lax.Precision.HIGHEST)

    ob = jnp.dot(q, bw_ref[...].T, preferred_element_type=f32,
                 precision=lax.Precision.HIGHEST) + bb_ref[...]
    rw = rw_ref[0]
    rw_c = jnp.dot(rw, t1, preferred_element_type=f32,
                 precision=lax.Precision.HIGHEST)
    rw_s = jnp.dot(rw, t2, preferred_element_type=f32,
                 precision=lax.Precision.HIGHEST)
    boxes = rw_c + ob * 0.125 * rw_s
    cx = jnp.dot(boxes, psel[0], preferred_element_type=f32,
                 precision=lax.Precision.HIGHEST)
    cy = jnp.dot(boxes, psel[1], preferred_element_type=f32,
                 precision=lax.Precision.HIGHEST)
    bw_sz = jnp.dot(boxes, psel[2], preferred_element_type=f32,
                 precision=lax.Precision.HIGHEST)
    bh_sz = jnp.dot(boxes, psel[3], preferred_element_type=f32,
                 precision=lax.Precision.HIGHEST)

    xoff = jnp.where(k_lane % 2 == 0, -0.25, 0.25).astype(f32)
    yoff = jnp.where(k_lane // 2 == 0, -0.25, 0.25).astype(f32)
    wlf = _by_level(l_lane, [64.0, 32.0, 16.0, 8.0], f32)
    hlf = wlf
    wli = _by_level(l_lane, [64, 32, 16, 8], i32)
    startl = _by_level(l_lane, [0, 4096, 5120, 5376], i32)

    gx = (cx + xoff * jnp.maximum(bw_sz, 0.0)) * rx_ref[0]
    gy = (cy + yoff * jnp.maximum(bh_sz, 0.0)) * ry_ref[0]
    x = gx * wlf - 0.5
    y = gy * hlf - 0.5
    x0 = jnp.floor(x)
    y0 = jnp.floor(y)
    fx = x - x0
    fy = y - y0

    boff = pl.program_id(0) * L2

    idx_refs = (i0_ref, i1_ref, i2_ref, i3_ref)
    w_refs = (w0_ref, w1_ref, w2_ref, w3_ref)
    corners = ((0.0, 0.0), (1.0, 0.0), (0.0, 1.0), (1.0, 1.0))
    for c, (dx, dy) in enumerate(corners):
        xc = x0 + dx
        yc = y0 + dy
        valid = ((xc >= 0.0) & (xc <= wlf - 1.0)
                 & (yc >= 0.0) & (yc <= hlf - 1.0))
        ii = jnp.clip(xc, 0.0, wlf - 1.0).astype(jnp.int32)
        jj = jnp.clip(yc, 0.0, hlf - 1.0).astype(jnp.int32)
        pos = startl + jj * wli + ii
        idx_refs[c][0] = (boff + pos) * NUM_HEAD + hlane
        wx = fx if dx == 1.0 else 1.0 - fx
        wy = fy if dy == 1.0 else 1.0 - fy
        w_refs[c][0] = wx * wy * valid.astype(jnp.float32)


def _oproj_body(x_ref, w_ref, b_ref, o_ref):
    o_ref[...] = jnp.dot(x_ref[...], w_ref[...].T,
                         preferred_element_type=jnp.float32,
                 precision=lax.Precision.HIGHEST) + b_ref[...]


_NC = 2
_NS = 16
_NW = _NC * _NS
_QPT = NQ // _NW
_QPC = 2
_NCHUNK = _QPT // _QPC


def _splat(vec, lane):
    dnums = lax.GatherDimensionNumbers(
        offset_dims=(), collapsed_slice_dims=(0,), start_index_map=(0,))
    idx = jnp.full((16, 1), lane, jnp.int32)
    return lax.gather(vec, idx, dnums, (1,),
                      mode=lax.GatherScatterMode.PROMISE_IN_BOUNDS)


_QPS = 16
_NSUPER = _QPT // _QPS
_CPS = _QPS // _QPC


def _sc_body(table, i0, i1, i2, i3, w0, w1, w2, w3, sp_hbm, lv_hbm,
             out_hbm, mask_hbm,
             idx_sv, wb_sv, sp_sv, lv_sv, rows_v, out_sv, mask_sv,
             sem0, sem1, msem0, msem1):
    wid = lax.axis_index("s") * _NC + lax.axis_index("c")
    q_tile = wid * _QPT
    idx_arrs = (i0, i1, i2, i3)
    wb_arrs = (w0, w1, w2, w3)
    sems = (sem0, sem1)
    msems = (msem0, msem1)

    def meta_copies(g, ms):
        qs0 = q_tile + g * _QPS
        cps = []
        for c in range(4):
            cps.append((idx_arrs[c].at[pl.ds(qs0, _QPS)],
                        idx_sv.at[pl.ds(ms * 64 + c * _QPS, _QPS)]))
            cps.append((wb_arrs[c].at[pl.ds(qs0 * 128, _QPS * 128)],
                        wb_sv.at[pl.ds(ms * 8192 + c * 2048, 2048)]))
        cps.append((sp_hbm.at[pl.ds(qs0 * 128, _QPS * 128)],
                    sp_sv.at[pl.ds(ms * 2048, 2048)]))
        cps.append((lv_hbm.at[pl.ds(qs0 * 128, _QPS * 128)],
                    lv_sv.at[pl.ds(ms * 2048, 2048)]))
        return cps

    def fire_meta(g, ms):
        for src, dst in meta_copies(g, ms):
            pltpu.async_copy(src, dst, msems[ms])

    def wait_meta(g, ms):
        for src, dst in meta_copies(g, ms):
            pltpu.make_async_copy(src, dst, msems[ms]).wait()

    def fire(j, slot, ms):
        for c in range(4):
            for q in range(_QPC):
                ql = j * _QPC + q
                pltpu.async_copy(
                    table.at[idx_sv.at[ms * 64 + c * _QPS + ql]],
                    rows_v.at[pl.ds(slot * 1024 + (q * 4 + c) * 128, 128)],
                    sems[slot])

    def wait_slot(slot):
        pltpu.make_async_copy(table.at[pl.ds(0, 1024)],
                              rows_v.at[pl.ds(slot * 1024, 1024)],
                              sems[slot]).wait()

    def compute(j, slot, ms):
        def t_body(t, c2):
            q = t // NUM_HEAD
            h = t % NUM_HEAD
            ql = j * _QPC + q
            lo = pl.ds(0, 16)
            hi = pl.ds(16, 16)
            spvec = sp_sv[pl.ds(ms * 2048 + ql * 128 + h * 16, 16)]
            lvvec = lv_sv[pl.ds(ms * 2048 + ql * 128 + h * 16, 16)]
            wvecs = [wb_sv[pl.ds(ms * 8192 + c * 2048 + ql * 128 + h * 16,
                                 16)]
                     for c in range(4)]
            out0 = jnp.zeros((16,), jnp.float32)
            out1 = jnp.zeros((16,), jnp.float32)
            for k in range(4):
                m0 = jnp.zeros((16,), jnp.float32)
                m1 = jnp.zeros((16,), jnp.float32)
                for l in range(4):
                    lk = l * 4 + k
                    s0 = jnp.zeros((16,), jnp.float32)
                    s1 = jnp.zeros((16,), jnp.float32)
                    for c in range(4):
                        rr = slot * 1024 + (q * 4 + c) * 128 + h * 16 + lk
                        w = _splat(wvecs[c], lk)
                        ve, vo = plsc.unpack(
                            rows_v[rr, :],
                            format=plsc.PackFormat.INTERLEAVED)
                        s0 = s0 + w * ve
                        s1 = s1 + w * vo
                    sp = _splat(spvec, lk)
                    lv = _splat(lvvec, lk)
                    out0 = out0 + sp * s0
                    out1 = out1 + sp * s1
                    m0 = m0 + lv * s0
                    m1 = m1 + lv * s1
                mrow = ql * 32 + k * 8 + h
                mask_sv[mrow, lo] = m0
                mask_sv[mrow, hi] = m1
            orow = ql * 8 + h
            out_sv[orow, lo] = out0
            out_sv[orow, hi] = out1
            return c2
        lax.fori_loop(0, _QPC * NUM_HEAD, t_body, 0)

    fire_meta(0, 0)

    def spair_body(gp, carry):
        for ms in range(2):
            g = 2 * gp + ms
            qs0 = q_tile + g * _QPS
            wait_meta(g, ms)

            @pl.when(g < _NSUPER - 1)
            def _():
                fire_meta(g + 1, 1 - ms)

            fire(0, 0, ms)

            def pair_body(p, c2):
                j0 = 2 * p
                fire(j0 + 1, 1, ms)
                wait_slot(0)
                compute(j0, 0, ms)

                @pl.when(p < _CPS // 2 - 1)
                def _():
                    fire(j0 + 2, 0, ms)

                wait_slot(1)
                compute(j0 + 1, 1, ms)
                return c2
            lax.fori_loop(0, _CPS // 2, pair_body, 0)

            pltpu.sync_copy(out_sv, out_hbm.at[pl.ds(qs0 * 8, _QPS * 8)])
            pltpu.sync_copy(mask_sv,
                            mask_hbm.at[pl.ds(qs0 * 32, _QPS * 32)])
        return carry

    lax.fori_loop(0, _NSUPER // 2, spair_body, 0)


@functools.lru_cache(maxsize=1)
def _build_sc():
    return pl.kernel(
        _sc_body,
        out_type=(jax.ShapeDtypeStruct((NITEM, HEAD_DIM), jnp.float32),
                  jax.ShapeDtypeStruct((NITEM * 4, HEAD_DIM), jnp.float32)),
        mesh=plsc.VectorSubcoreMesh(core_axis_name="c", subcore_axis_name="s",
                                    num_cores=_NC, num_subcores=_NS),
        compiler_params=pltpu.CompilerParams(use_tc_tiling_on_sc=False,
                                             needs_layout_passes=False),
        scratch_types=[
            pltpu.VMEM((2 * 4 * _QPS, 128), jnp.int32),
            pltpu.VMEM((2 * 4 * _QPS * 128,), jnp.float32),
            pltpu.VMEM((2 * _QPS * 128,), jnp.float32),
            pltpu.VMEM((2 * _QPS * 128,), jnp.float32),
            pltpu.VMEM((2 * 1024, HEAD_DIM), jnp.bfloat16),
            pltpu.VMEM((_QPS * 8, HEAD_DIM), jnp.float32),
            pltpu.VMEM((_QPS * 32, HEAD_DIM), jnp.float32),
            pltpu.SemaphoreType.DMA,
            pltpu.SemaphoreType.DMA,
            pltpu.SemaphoreType.DMA,
            pltpu.SemaphoreType.DMA,
        ],
    )


def _sc_call(table, idxs, wbs, sp_flat, lv_flat):
    return _build_sc()(table, *idxs, *wbs, sp_flat, lv_flat)


def kernel(query, value, v_shape, v_mask, v_start_index, v_valid_ratios,
           ref_windows, value_proj_w, value_proj_b, out_proj_w, out_proj_b,
           box_w, box_b, attn_w, attn_b):
    f32 = jnp.float32

    maskf = (1.0 - v_mask.astype(f32)).reshape(B * L2, 1)
    vproj = pl.pallas_call(
        _vproj_body,
        grid=(8,),
        in_specs=[
            pl.BlockSpec((B * L2 // 8, D_MODEL), lambda i: (i, 0)),
            pl.BlockSpec((D_MODEL, D_MODEL), lambda i: (0, 0)),
            pl.BlockSpec((1, D_MODEL), lambda i: (0, 0)),
            pl.BlockSpec((B * L2 // 8, 1), lambda i: (i, 0)),
        ],
        out_specs=pl.BlockSpec((B * L2 // 8, D_MODEL), lambda i: (i, 0)),
        out_shape=jax.ShapeDtypeStruct((B * L2, D_MODEL), jnp.bfloat16),
    )(value.reshape(B * L2, D_MODEL), value_proj_w,
      value_proj_b.reshape(1, D_MODEL), maskf)
    table = vproj.reshape(B * L2 * NUM_HEAD, HEAD_DIM)

    hd_perm = jnp.asarray(
        np.concatenate([h * 32 + np.concatenate([np.arange(0, 32, 2),
                                                 np.arange(1, 32, 2)])
                        for h in range(NUM_HEAD)]).astype(np.int32))
    out_proj_w_p = out_proj_w[:, hd_perm]

    r = v_valid_ratios[:, 0, 0, :, 0, :]
    rx = jnp.broadcast_to(r[:, None, :, None, 0], (B, 8, 4, 4)).reshape(B, 1, 128)
    ry = jnp.broadcast_to(r[:, None, :, None, 1], (B, 8, 4, 4)).reshape(B, 1, 128)
    qout = pl.pallas_call(
        _qside_body,
        grid=(B,),
        in_specs=[
            pl.BlockSpec((1, L1, D_MODEL), lambda b: (b, 0, 0)),
            pl.BlockSpec((1, L1, 4), lambda b: (b, 0, 0)),
            pl.BlockSpec((128, D_MODEL), lambda b: (0, 0)),
            pl.BlockSpec((1, 128), lambda b: (0, 0)),
            pl.BlockSpec((128, D_MODEL), lambda b: (0, 0)),
            pl.BlockSpec((1, 128), lambda b: (0, 0)),
            pl.BlockSpec((1, 1, 128), lambda b: (b, 0, 0)),
            pl.BlockSpec((1, 1, 128), lambda b: (b, 0, 0)),
        ],
        out_specs=[pl.BlockSpec((1, L1, 128), lambda b: (b, 0, 0))] * 10,
        out_shape=[jax.ShapeDtypeStruct((B, L1, 128), f32),
                   jax.ShapeDtypeStruct((B, L1, 128), f32)]
                  + [jax.ShapeDtypeStruct((B, L1, 128), jnp.int32)] * 4
                  + [jax.ShapeDtypeStruct((B, L1, 128), f32)] * 4,
    )(query, ref_windows, attn_w, attn_b.reshape(1, 128),
      box_w, box_b.reshape(1, 128), rx, ry)
    spatial_l, level_l, i0, i1, i2, i3, w0, w1, w2, w3 = qout

    out_s, mask_s = _sc_call(
        table,
        [i.reshape(NQ, 128) for i in (i0, i1, i2, i3)],
        [w.reshape(-1) for w in (w0, w1, w2, w3)],
        spatial_l.reshape(-1), level_l.reshape(-1))

    def oproj(x, rows_per_block, nblocks):
        return pl.pallas_call(
            _oproj_body,
            grid=(nblocks,),
            in_specs=[
                pl.BlockSpec((rows_per_block, D_MODEL), lambda i: (i, 0)),
                pl.BlockSpec((D_MODEL, D_MODEL), lambda i: (0, 0)),
                pl.BlockSpec((1, D_MODEL), lambda i: (0, 0)),
            ],
            out_specs=pl.BlockSpec((rows_per_block, D_MODEL), lambda i: (i, 0)),
            out_shape=jax.ShapeDtypeStruct((x.shape[0], D_MODEL), f32),
        )(x, out_proj_w_p, out_proj_b.reshape(1, D_MODEL))

    out = oproj(out_s.reshape(NQ, D_MODEL), 2048, 2).reshape(B, L1, D_MODEL)
    mask_out = oproj(mask_s.reshape(NQ * 4, D_MODEL), 2048, 8)
    mask_out = mask_out.reshape(B, L1, KK, D_MODEL)

    spatial = spatial_l.reshape(B, L1, NUM_HEAD, NUM_LEVEL, KK)
    level = level_l.reshape(B, L1, NUM_HEAD, NUM_LEVEL, KK)
    return (out, mask_out, spatial, level)
